# baseline (speedup 1.0000x reference)
import jax
import jax.numpy as jnp
from jax.experimental import pallas as pl
from jax.experimental.pallas import tpu as pltpu

P = 4096
B = 6
F_IN = 960
D_PAD = 1024
NCOL = 12
G = 16
NP = B * G
PB1 = 2048
NP1 = P // PB1


def _proj_kernel(d_ref, wsh_ref, t_ref):
    b = pl.program_id(1)
    part = jnp.dot(d_ref[...], wsh_ref[b], preferred_element_type=jnp.float32)

    @pl.when(b == 0)
    def _init():
        t_ref[...] = part

    @pl.when(b > 0)
    def _accum():
        t_ref[...] = t_ref[...] + part


def _agg_kernel(adj_ref, tq_ref, tp_ref, bias_ref, pm_ref, rgb_ref):
    acc = jnp.dot(adj_ref[...], tq_ref[...], preferred_element_type=jnp.float32)
    tp = tp_ref[...]
    bz = bias_ref[...]
    for b in range(B):
        g = b * G
        pm_ref[b] = jnp.tanh(acc[:, g:g + 3] + tp[:, g + 6:g + 9] + bz[:, g:g + 3])
        rgb_ref[b] = jax.nn.sigmoid(
            acc[:, g + 3:g + 6] + tp[:, g + 9:g + 12] + bz[:, g + 3:g + 6]
        )


def kernel(embeddings, ref, adj, W_d, Wl_d, b_d, W_r, Wl_r, b_r):
    d_pad = jnp.concatenate(
        [embeddings,
         jnp.broadcast_to(ref, (B, P, 3)),
         jnp.zeros((B, P, D_PAD - F_IN - 3), jnp.float32)],
        axis=2,
    ).astype(jnp.bfloat16).reshape(B * P, D_PAD)

    W_all = jnp.concatenate([W_d, W_r, Wl_d, Wl_r], axis=1)
    W_pad = jnp.concatenate(
        [W_all, jnp.zeros((D_PAD - 963, NCOL), jnp.float32)], axis=0
    )
    w_sh = jnp.stack([
        jnp.concatenate(
            [jnp.zeros((D_PAD, b * G), jnp.float32), W_pad,
             jnp.zeros((D_PAD, NP - b * G - NCOL), jnp.float32)], axis=1)
        for b in range(B)
    ]).astype(jnp.bfloat16)
    bias = jnp.tile(
        jnp.concatenate([b_d, b_r, jnp.zeros((G - 6,), jnp.float32)]), B
    ).reshape(1, NP)

    tpk = pl.pallas_call(
        _proj_kernel,
        grid=(NP1, B),
        in_specs=[
            pl.BlockSpec((PB1, D_PAD), lambda p, b: (b * NP1 + p, 0)),
            pl.BlockSpec((B, D_PAD, NP), lambda p, b: (0, 0, 0)),
        ],
        out_specs=pl.BlockSpec((PB1, NP), lambda p, b: (p, 0)),
        out_shape=jax.ShapeDtypeStruct((P, NP), jnp.float32),
        compiler_params=pltpu.CompilerParams(
            dimension_semantics=("arbitrary", "arbitrary"),
        ),
    )(d_pad, w_sh)

    PBLK = 512
    npb = P // PBLK
    pm, rgb = pl.pallas_call(
        _agg_kernel,
        grid=(npb,),
        in_specs=[
            pl.BlockSpec((PBLK, P), lambda p: (p, 0)),
            pl.BlockSpec((P, NP), lambda p: (0, 0)),
            pl.BlockSpec((PBLK, NP), lambda p: (p, 0)),
            pl.BlockSpec((1, NP), lambda p: (0, 0)),
        ],
        out_specs=[
            pl.BlockSpec((B, PBLK, 3), lambda p: (0, p, 0)),
            pl.BlockSpec((B, PBLK, 3), lambda p: (0, p, 0)),
        ],
        out_shape=[
            jax.ShapeDtypeStruct((B, P, 3), jnp.float32),
            jax.ShapeDtypeStruct((B, P, 3), jnp.float32),
        ],
    )(adj, tpk, tpk, bias)
    return pm, rgb


# speedup vs baseline: 1.2077x; 1.2077x over previous
import jax
import jax.numpy as jnp
from jax.experimental import pallas as pl
from jax.experimental.pallas import tpu as pltpu

P = 4096
B = 6
F_IN = 960
NCOL = 12
G = 16
NP = B * G
PB1 = 512
NP1 = P // PB1
PBLK = 512
NPB = P // PBLK


def _fused_kernel(emb_ref, refc_ref, w_emb_ref, w_ref_ref, adj_ref, bias_ref,
                  pm_ref, rgb_ref, tpk_scr):
    i = pl.program_id(0)

    @pl.when(i < NP1)
    def _project():
        e = emb_ref[...].reshape(B * PB1, F_IN)
        t_all = jnp.dot(e, w_emb_ref[...], preferred_element_type=jnp.float32)
        rw = jnp.dot(refc_ref[...], w_ref_ref[...], preferred_element_type=jnp.float32)
        row = i * PB1
        for b in range(B):
            tpk_scr[pl.ds(row, PB1), b * G:b * G + NCOL] = (
                t_all[b * PB1:(b + 1) * PB1, :] + rw
            )

    @pl.when(i >= NP1)
    def _aggregate():
        acc = jnp.dot(adj_ref[...], tpk_scr[...], preferred_element_type=jnp.float32)
        tp = tpk_scr[pl.ds((i - NP1) * PBLK, PBLK), :]
        bz = bias_ref[...]
        for b in range(B):
            g = b * G
            pm_ref[b] = jnp.tanh(acc[:, g:g + 3] + tp[:, g + 6:g + 9] + bz[:, g:g + 3])
            rgb_ref[b] = jax.nn.sigmoid(
                acc[:, g + 3:g + 6] + tp[:, g + 9:g + 12] + bz[:, g + 3:g + 6]
            )


def kernel(embeddings, ref, adj, W_d, Wl_d, b_d, W_r, Wl_r, b_r):
    refc = ref.reshape(P, 3)
    W_all = jnp.concatenate([W_d, W_r, Wl_d, Wl_r], axis=1)
    W_emb = W_all[:F_IN]
    W_ref = W_all[F_IN:]
    bias = jnp.tile(
        jnp.concatenate([b_d, b_r, jnp.zeros((G - 6,), jnp.float32)]), B
    ).reshape(1, NP)

    pm, rgb = pl.pallas_call(
        _fused_kernel,
        grid=(NP1 + NPB,),
        in_specs=[
            pl.BlockSpec((B, PB1, F_IN),
                         lambda i: (0, jnp.minimum(i, NP1 - 1), 0)),
            pl.BlockSpec((PB1, 3), lambda i: (jnp.minimum(i, NP1 - 1), 0)),
            pl.BlockSpec((F_IN, NCOL), lambda i: (0, 0)),
            pl.BlockSpec((3, NCOL), lambda i: (0, 0)),
            pl.BlockSpec((PBLK, P),
                         lambda i: (jnp.clip(i - NP1, 0, NPB - 1), 0)),
            pl.BlockSpec((1, NP), lambda i: (0, 0)),
        ],
        out_specs=[
            pl.BlockSpec((B, PBLK, 3), lambda i: (0, jnp.maximum(i - NP1, 0), 0)),
            pl.BlockSpec((B, PBLK, 3), lambda i: (0, jnp.maximum(i - NP1, 0), 0)),
        ],
        out_shape=[
            jax.ShapeDtypeStruct((B, P, 3), jnp.float32),
            jax.ShapeDtypeStruct((B, P, 3), jnp.float32),
        ],
        scratch_shapes=[pltpu.VMEM((P, NP), jnp.float32)],
        compiler_params=pltpu.CompilerParams(
            dimension_semantics=("arbitrary",),
        ),
    )(embeddings, refc, W_emb, W_ref, adj, bias)
    return pm, rgb
